# baseline (speedup 1.0000x reference)
import functools

import jax
import jax.numpy as jnp
from jax import lax
from jax.experimental import pallas as pl
from jax.experimental.pallas import tpu as pltpu
from jax.experimental.pallas import tpu_sc as plsc

_NC = 2
_NS = 16
_NW = _NC * _NS
_L = 16
_ROWS = 8
_NBUF = 2


@functools.lru_cache(maxsize=None)
def _make_kernel(n_seg: int, h: int, w: int):
    rows_per_w = h // _NW
    n_chunks = rows_per_w // _ROWS
    chunk_elems = _ROWS * w
    mesh = plsc.VectorSubcoreMesh(core_axis_name="c", subcore_axis_name="s")

    @functools.partial(
        pl.kernel,
        mesh=mesh,
        out_type=jax.ShapeDtypeStruct((h, w), jnp.float32),
        compiler_params=pltpu.CompilerParams(
            needs_layout_passes=False,
            disable_bounds_checks=True,
            disable_semaphore_checks=True,
            skip_device_barrier=True,
        ),
        scratch_types=[
            pltpu.VMEM((n_seg,), jnp.float32),
            pltpu.VMEM((_ROWS, w), jnp.int32),
            pltpu.VMEM((_ROWS, w), jnp.int32),
            pltpu.VMEM((_ROWS, w), jnp.float32),
            pltpu.VMEM((_ROWS, w), jnp.float32),
            pltpu.SemaphoreType.DMA,
            pltpu.SemaphoreType.DMA,
            pltpu.SemaphoreType.DMA,
            pltpu.SemaphoreType.DMA,
        ],
    )
    def gather_kernel(inp_hbm, idx_hbm, out_hbm, table_v, idx_v0, idx_v1,
                      out_v0, out_v1, isem0, isem1, osem0, osem1):
        idx_bufs = (idx_v0, idx_v1)
        out_bufs = (out_v0, out_v1)
        isems = (isem0, isem1)
        osems = (osem0, osem1)
        wid = lax.axis_index("s") * _NC + lax.axis_index("c")
        row0 = wid * rows_per_w
        pltpu.sync_copy(inp_hbm, table_v)

        def start_idx(g):
            return pltpu.async_copy(
                idx_hbm.at[pl.ds(row0 + g * _ROWS, _ROWS), :],
                idx_bufs[g % _NBUF], isems[g % _NBUF])

        def start_out(g):
            return pltpu.async_copy(
                out_bufs[g % _NBUF],
                out_hbm.at[pl.ds(row0 + g * _ROWS, _ROWS), :],
                osems[g % _NBUF])

        idx_copies = {0: start_idx(0)}
        out_copies = {}
        for g in range(n_chunks):
            if g + 1 < n_chunks:
                idx_copies[g + 1] = start_idx(g + 1)
            idx_copies[g].wait()
            if g >= _NBUF:
                out_copies[g - _NBUF].wait()
            ib = idx_bufs[g % _NBUF]
            ob = out_bufs[g % _NBUF]
            for r in range(_ROWS):
                @plsc.parallel_loop(0, w, step=_L, unroll=8)
                def _body(i, ib=ib, ob=ob, r=r):
                    iv = ib[r, pl.ds(i, _L)]
                    ob[r, pl.ds(i, _L)] = plsc.load_gather(table_v, [iv])

            out_copies[g] = start_out(g)
        for g in range(max(0, n_chunks - _NBUF), n_chunks):
            out_copies[g].wait()

    return gather_kernel


def kernel(input, segm_img):
    n_seg = input.shape[0]
    h, w = segm_img.shape
    return _make_kernel(n_seg, h, w)(input, segm_img)


# speedup vs baseline: 1.0603x; 1.0603x over previous
import functools

import jax
import jax.numpy as jnp
from jax import lax
from jax.experimental import pallas as pl
from jax.experimental.pallas import tpu as pltpu
from jax.experimental.pallas import tpu_sc as plsc

_NC = 2
_NS = 16
_NW = _NC * _NS
_L = 16
_ROWS = 8
_NBUF = 2


@functools.lru_cache(maxsize=None)
def _make_kernel(n_seg: int, h: int, w: int):
    rows_per_w = h // _NW
    n_chunks = rows_per_w // _ROWS
    chunk_elems = _ROWS * w
    mesh = plsc.VectorSubcoreMesh(core_axis_name="c", subcore_axis_name="s")

    @functools.partial(
        pl.kernel,
        mesh=mesh,
        out_type=jax.ShapeDtypeStruct((h, w), jnp.float32),
        compiler_params=pltpu.CompilerParams(needs_layout_passes=False),
        scratch_types=[
            pltpu.VMEM((n_seg,), jnp.float32),
            pltpu.VMEM((_ROWS, w), jnp.int32),
            pltpu.VMEM((_ROWS, w), jnp.int32),
            pltpu.VMEM((_ROWS, w), jnp.float32),
            pltpu.VMEM((_ROWS, w), jnp.float32),
            pltpu.SemaphoreType.DMA,
            pltpu.SemaphoreType.DMA,
            pltpu.SemaphoreType.DMA,
            pltpu.SemaphoreType.DMA,
        ],
    )
    def gather_kernel(inp_hbm, idx_hbm, out_hbm, table_v, idx_v0, idx_v1,
                      out_v0, out_v1, isem0, isem1, osem0, osem1):
        idx_bufs = (idx_v0, idx_v1)
        out_bufs = (out_v0, out_v1)
        isems = (isem0, isem1)
        osems = (osem0, osem1)
        wid = lax.axis_index("s") * _NC + lax.axis_index("c")
        row0 = wid * rows_per_w
        pltpu.sync_copy(inp_hbm, table_v)

        def start_idx(g):
            return pltpu.async_copy(
                idx_hbm.at[pl.ds(row0 + g * _ROWS, _ROWS), :],
                idx_bufs[g % _NBUF], isems[g % _NBUF])

        def start_out(g):
            return pltpu.async_copy(
                out_bufs[g % _NBUF],
                out_hbm.at[pl.ds(row0 + g * _ROWS, _ROWS), :],
                osems[g % _NBUF])

        idx_copies = {0: start_idx(0)}
        out_copies = {}
        for g in range(n_chunks):
            if g + 1 < n_chunks:
                idx_copies[g + 1] = start_idx(g + 1)
            idx_copies[g].wait()
            if g >= _NBUF:
                out_copies[g - _NBUF].wait()
            ib = idx_bufs[g % _NBUF]
            ob = out_bufs[g % _NBUF]

            @plsc.parallel_loop(0, w, step=_L, unroll=2)
            def _body(i, ib=ib, ob=ob):
                for r in range(_ROWS):
                    iv = ib[r, pl.ds(i, _L)]
                    ob[r, pl.ds(i, _L)] = plsc.load_gather(table_v, [iv])

            out_copies[g] = start_out(g)
        for g in range(max(0, n_chunks - _NBUF), n_chunks):
            out_copies[g].wait()

    return gather_kernel


def kernel(input, segm_img):
    n_seg = input.shape[0]
    h, w = segm_img.shape
    return _make_kernel(n_seg, h, w)(input, segm_img)
